# Initial kernel scaffold; baseline (speedup 1.0000x reference)
import jax
import jax.numpy as jnp
from jax.experimental import pallas as pl


def kernel(f, provenance, size):
    raise NotImplementedError("write your pallas kernel here")


# speedup vs baseline: 81.5245x; 81.5245x over previous
import functools

import jax
import jax.numpy as jnp
from jax import lax
from jax.experimental import pallas as pl
from jax.experimental.pallas import tpu as pltpu
from jax.experimental.pallas import tpu_sc as plsc

_NUM_CORES = 2
_NUM_SUBCORES = 16
_NW = _NUM_CORES * _NUM_SUBCORES
_L = 16


def _build_sc_unpool(B, C, Hp, Wp):
    planes = B * C
    assert planes % _NW == 0
    ppw = planes // _NW
    W2 = 2 * Wp
    pair = 2 * W2
    CH = 32
    assert Hp % CH == 0
    cpp = Hp // CH
    gpr = Wp // _L
    in_words = CH * Wp
    out_words = CH * pair

    mesh = plsc.VectorSubcoreMesh(core_axis_name="c", subcore_axis_name="s")

    @functools.partial(
        pl.kernel,
        out_type=jax.ShapeDtypeStruct((planes * Hp * Wp * 4,), jnp.float32),
        mesh=mesh,
        compiler_params=pltpu.CompilerParams(needs_layout_passes=False),
        scratch_types=[
            pltpu.VMEM((in_words,), jnp.float32),
            pltpu.VMEM((in_words,), jnp.int32),
            pltpu.VMEM((out_words,), jnp.float32),
        ],
    )
    def unpool(f_hbm, p_hbm, out_hbm, fbuf, pbuf, obuf):
        wid = lax.axis_index("s") * _NUM_CORES + lax.axis_index("c")
        ii2 = 2 * lax.iota(jnp.int32, _L)
        zero = jnp.zeros((_L,), jnp.float32)

        def chunk_body(p_loc, cc):
            row0 = (wid * ppw + p_loc) * Hp + cc * CH
            i_base = cc * CH
            pltpu.sync_copy(f_hbm.at[pl.ds(row0 * Wp, in_words)], fbuf)
            pltpu.sync_copy(p_hbm.at[pl.ds(row0 * Wp, in_words)], pbuf)

            def row_body(k, _):
                qc = pair * (i_base + k)
                rowoff = pair * k
                for t in range(gpr):
                    off = k * Wp + t * _L
                    pv = pbuf[pl.ds(off, _L)]
                    fv = fbuf[pl.ds(off, _L)]
                    q = pv - qc
                    m_even_row = q < W2
                    m_even_col = (q & 1) == 0
                    r0 = jnp.where(m_even_row, fv, zero)
                    r1 = fv - r0
                    c00 = jnp.where(m_even_col, r0, zero)
                    c01 = r0 - c00
                    c10 = jnp.where(m_even_col, r1, zero)
                    c11 = r1 - c10
                    base = rowoff + 2 * _L * t + ii2
                    plsc.store_scatter(obuf, [base], c00)
                    plsc.store_scatter(obuf, [base + 1], c01)
                    plsc.store_scatter(obuf, [base + W2], c10)
                    plsc.store_scatter(obuf, [base + W2 + 1], c11)
                return 0

            lax.fori_loop(0, CH, row_body, 0)
            pltpu.sync_copy(obuf, out_hbm.at[pl.ds(row0 * pair, out_words)])
            return 0

        def plane_body(p_loc, _):
            lax.fori_loop(0, cpp, lambda cc, _: chunk_body(p_loc, cc), 0)
            return 0

        lax.fori_loop(0, ppw, plane_body, 0)

    return unpool


def kernel(f, provenance, size):
    B, C, Hp, Wp = f.shape
    unpool = _build_sc_unpool(B, C, Hp, Wp)
    out_flat = unpool(f.reshape(-1), provenance.reshape(-1))
    return out_flat.reshape(B, C, 2 * Hp, 2 * Wp)


# speedup vs baseline: 107.6685x; 1.3207x over previous
import functools

import jax
import jax.numpy as jnp
from jax import lax
from jax.experimental import pallas as pl
from jax.experimental.pallas import tpu as pltpu
from jax.experimental.pallas import tpu_sc as plsc

_NUM_CORES = 2
_NUM_SUBCORES = 16
_NW = _NUM_CORES * _NUM_SUBCORES
_L = 16


def _build_sc_unpool(B, C, Hp, Wp):
    planes = B * C
    assert planes % _NW == 0
    ppw = planes // _NW
    W2 = 2 * Wp
    pair = 2 * W2
    CH = 32
    assert Hp % CH == 0
    cpp = Hp // CH
    gpr = Wp // _L
    in_words = CH * Wp
    out_words = CH * pair
    rows_pw = ppw * Hp
    nch = ppw * cpp
    npair = nch // 2
    assert nch % 2 == 0 and npair >= 3

    mesh = plsc.VectorSubcoreMesh(core_axis_name="c", subcore_axis_name="s")

    @functools.partial(
        pl.kernel,
        out_type=jax.ShapeDtypeStruct((planes * Hp * Wp * 4,), jnp.float32),
        mesh=mesh,
        compiler_params=pltpu.CompilerParams(needs_layout_passes=False),
        scratch_types=[
            pltpu.VMEM((in_words,), jnp.float32),
            pltpu.VMEM((in_words,), jnp.float32),
            pltpu.VMEM((in_words,), jnp.int32),
            pltpu.VMEM((in_words,), jnp.int32),
            pltpu.VMEM((out_words,), jnp.float32),
            pltpu.VMEM((out_words,), jnp.float32),
            pltpu.SemaphoreType.DMA,
            pltpu.SemaphoreType.DMA,
            pltpu.SemaphoreType.DMA,
            pltpu.SemaphoreType.DMA,
            pltpu.SemaphoreType.DMA,
            pltpu.SemaphoreType.DMA,
        ],
    )
    def unpool(f_hbm, p_hbm, out_hbm,
               fb0, fb1, pb0, pb1, ob0, ob1, sf0, sf1, sp0, sp1, so0, so1):
        wid = lax.axis_index("s") * _NUM_CORES + lax.axis_index("c")
        ii2 = 2 * lax.iota(jnp.int32, _L)
        zero = jnp.zeros((_L,), jnp.float32)
        bufsets = ((fb0, pb0, ob0, sf0, sp0, so0),
                   (fb1, pb1, ob1, sf1, sp1, so1))

        def row0_of(ch):
            return wid * rows_pw + ch * CH

        def in_copies(ch, fb, pb, sf, sp):
            row0 = row0_of(ch)
            src = pl.ds(row0 * Wp, in_words)
            return (pltpu.make_async_copy(f_hbm.at[src], fb, sf),
                    pltpu.make_async_copy(p_hbm.at[src], pb, sp))

        def start_in(ch, fb, pb, sf, sp):
            for c in in_copies(ch, fb, pb, sf, sp):
                c.start()

        def wait_in(ch, fb, pb, sf, sp):
            for c in in_copies(ch, fb, pb, sf, sp):
                c.wait()

        def out_copy(ch, ob, so):
            row0 = row0_of(ch)
            return pltpu.make_async_copy(
                ob, out_hbm.at[pl.ds(row0 * pair, out_words)], so)

        def compute(ch, fb, pb, ob):
            i_base = lax.rem(ch, cpp) * CH

            def row_body(k, _):
                qc = pair * (i_base + k)
                rowoff = pair * k
                for t in range(gpr):
                    off = k * Wp + t * _L
                    pv = pb[pl.ds(off, _L)]
                    fv = fb[pl.ds(off, _L)]
                    q = pv - qc
                    m_even_row = q < W2
                    m_even_col = (q & 1) == 0
                    r0 = jnp.where(m_even_row, fv, zero)
                    r1 = fv - r0
                    c00 = jnp.where(m_even_col, r0, zero)
                    c01 = r0 - c00
                    c10 = jnp.where(m_even_col, r1, zero)
                    c11 = r1 - c10
                    base = rowoff + 2 * _L * t + ii2
                    plsc.store_scatter(ob, [base], c00)
                    plsc.store_scatter(ob, [base + 1], c01)
                    plsc.store_scatter(ob, [base + W2], c10)
                    plsc.store_scatter(ob, [base + W2 + 1], c11)
                return 0

            lax.fori_loop(0, CH, row_body, 0)

        for b, (fb, pb, ob, sf, sp, so) in enumerate(bufsets):
            start_in(b, fb, pb, sf, sp)

        for b, (fb, pb, ob, sf, sp, so) in enumerate(bufsets):
            wait_in(b, fb, pb, sf, sp)
            compute(b, fb, pb, ob)
            out_copy(b, ob, so).start()
            start_in(b + 2, fb, pb, sf, sp)

        def mid(g, _):
            for b, (fb, pb, ob, sf, sp, so) in enumerate(bufsets):
                ch = 2 * g + b
                wait_in(ch, fb, pb, sf, sp)
                out_copy(ch - 2, ob, so).wait()
                compute(ch, fb, pb, ob)
                out_copy(ch, ob, so).start()
                start_in(ch + 2, fb, pb, sf, sp)
            return 0

        lax.fori_loop(1, npair - 1, mid, 0)

        for b, (fb, pb, ob, sf, sp, so) in enumerate(bufsets):
            ch = nch - 2 + b
            wait_in(ch, fb, pb, sf, sp)
            out_copy(ch - 2, ob, so).wait()
            compute(ch, fb, pb, ob)
            out_copy(ch, ob, so).start()

        for b, (fb, pb, ob, sf, sp, so) in enumerate(bufsets):
            out_copy(nch - 2 + b, ob, so).wait()

    return unpool


def kernel(f, provenance, size):
    B, C, Hp, Wp = f.shape
    unpool = _build_sc_unpool(B, C, Hp, Wp)
    out_flat = unpool(f.reshape(-1), provenance.reshape(-1))
    return out_flat.reshape(B, C, 2 * Hp, 2 * Wp)


# speedup vs baseline: 127.8379x; 1.1873x over previous
import functools

import jax
import jax.numpy as jnp
from jax import lax
from jax.experimental import pallas as pl
from jax.experimental.pallas import tpu as pltpu
from jax.experimental.pallas import tpu_sc as plsc

_NUM_CORES = 2
_NUM_SUBCORES = 16
_NW = _NUM_CORES * _NUM_SUBCORES
_L = 16


def _build_sc_unpool(B, C, Hp, Wp):
    planes = B * C
    assert planes % _NW == 0
    ppw = planes // _NW
    W2 = 2 * Wp
    pair = 2 * W2
    CH = 32
    assert Hp % CH == 0
    cpp = Hp // CH
    gpr = Wp // _L
    in_words = CH * Wp
    out_words = CH * pair
    rows_pw = ppw * Hp
    nch = ppw * cpp
    npair = nch // 2
    assert nch % 2 == 0 and npair >= 3

    mesh = plsc.VectorSubcoreMesh(core_axis_name="c", subcore_axis_name="s")

    @functools.partial(
        pl.kernel,
        out_type=jax.ShapeDtypeStruct((planes * Hp * Wp * 4,), jnp.float32),
        mesh=mesh,
        compiler_params=pltpu.CompilerParams(needs_layout_passes=False),
        scratch_types=[
            pltpu.VMEM((in_words,), jnp.float32),
            pltpu.VMEM((in_words,), jnp.float32),
            pltpu.VMEM((in_words,), jnp.int32),
            pltpu.VMEM((in_words,), jnp.int32),
            pltpu.VMEM((out_words + pair,), jnp.float32),
            pltpu.VMEM((out_words + pair,), jnp.float32),
            pltpu.SemaphoreType.DMA,
            pltpu.SemaphoreType.DMA,
            pltpu.SemaphoreType.DMA,
            pltpu.SemaphoreType.DMA,
            pltpu.SemaphoreType.DMA,
            pltpu.SemaphoreType.DMA,
        ],
    )
    def unpool(f_hbm, p_hbm, out_hbm,
               fb0, fb1, pb0, pb1, ob0, ob1, sf0, sf1, sp0, sp1, so0, so1):
        wid = lax.axis_index("s") * _NUM_CORES + lax.axis_index("c")
        ii2 = 2 * lax.iota(jnp.int32, _L)
        zero = jnp.zeros((_L,), jnp.float32)
        bvecs = [2 * _L * t + ii2 for t in range(gpr)]
        bvecs1 = [b + 1 for b in bvecs]
        bufsets = ((fb0, pb0, ob0, sf0, sp0, so0),
                   (fb1, pb1, ob1, sf1, sp1, so1))

        def row0_of(ch):
            return wid * rows_pw + ch * CH

        def in_copies(ch, fb, pb, sf, sp):
            row0 = row0_of(ch)
            src = pl.ds(row0 * Wp, in_words)
            return (pltpu.make_async_copy(f_hbm.at[src], fb, sf),
                    pltpu.make_async_copy(p_hbm.at[src], pb, sp))

        def start_in(ch, fb, pb, sf, sp):
            for c in in_copies(ch, fb, pb, sf, sp):
                c.start()

        def wait_in(ch, fb, pb, sf, sp):
            for c in in_copies(ch, fb, pb, sf, sp):
                c.wait()

        def out_copy(ch, ob, so):
            row0 = row0_of(ch)
            return pltpu.make_async_copy(
                ob.at[pl.ds(0, out_words)],
                out_hbm.at[pl.ds(row0 * pair, out_words)], so)

        def compute(ch, fb, pb, ob):
            i_base = lax.rem(ch, cpp) * CH

            @plsc.parallel_loop(0, CH, unroll=2)
            def row_body(k):
                qc = pair * (i_base + k)
                rowoff = pair * k
                ob_e = ob.at[pl.ds(rowoff, pair)]
                ob_o = ob.at[pl.ds(rowoff + W2, pair)]
                for t in range(gpr):
                    off = k * Wp + t * _L
                    pv = pb[pl.ds(off, _L)]
                    fv = fb[pl.ds(off, _L)]
                    q = pv - qc
                    m_even_row = q < W2
                    m_even_col = (q & 1) == 0
                    r0 = jnp.where(m_even_row, fv, zero)
                    r1 = fv - r0
                    c00 = jnp.where(m_even_col, r0, zero)
                    c01 = r0 - c00
                    c10 = jnp.where(m_even_col, r1, zero)
                    c11 = r1 - c10
                    plsc.store_scatter(ob_e, [bvecs[t]], c00)
                    plsc.store_scatter(ob_e, [bvecs1[t]], c01)
                    plsc.store_scatter(ob_o, [bvecs[t]], c10)
                    plsc.store_scatter(ob_o, [bvecs1[t]], c11)

        for b, (fb, pb, ob, sf, sp, so) in enumerate(bufsets):
            start_in(b, fb, pb, sf, sp)

        for b, (fb, pb, ob, sf, sp, so) in enumerate(bufsets):
            wait_in(b, fb, pb, sf, sp)
            compute(b, fb, pb, ob)
            out_copy(b, ob, so).start()
            start_in(b + 2, fb, pb, sf, sp)

        def mid(g, _):
            for b, (fb, pb, ob, sf, sp, so) in enumerate(bufsets):
                ch = 2 * g + b
                wait_in(ch, fb, pb, sf, sp)
                out_copy(ch - 2, ob, so).wait()
                compute(ch, fb, pb, ob)
                out_copy(ch, ob, so).start()
                start_in(ch + 2, fb, pb, sf, sp)
            return 0

        lax.fori_loop(1, npair - 1, mid, 0)

        for b, (fb, pb, ob, sf, sp, so) in enumerate(bufsets):
            ch = nch - 2 + b
            wait_in(ch, fb, pb, sf, sp)
            out_copy(ch - 2, ob, so).wait()
            compute(ch, fb, pb, ob)
            out_copy(ch, ob, so).start()

        for b, (fb, pb, ob, sf, sp, so) in enumerate(bufsets):
            out_copy(nch - 2 + b, ob, so).wait()

    return unpool


def kernel(f, provenance, size):
    B, C, Hp, Wp = f.shape
    unpool = _build_sc_unpool(B, C, Hp, Wp)
    out_flat = unpool(f.reshape(-1), provenance.reshape(-1))
    return out_flat.reshape(B, C, 2 * Hp, 2 * Wp)


# speedup vs baseline: 127.9411x; 1.0008x over previous
import functools

import jax
import jax.numpy as jnp
from jax import lax
from jax.experimental import pallas as pl
from jax.experimental.pallas import tpu as pltpu
from jax.experimental.pallas import tpu_sc as plsc

_NUM_CORES = 2
_NUM_SUBCORES = 16
_NW = _NUM_CORES * _NUM_SUBCORES
_L = 16


def _build_sc_unpool(B, C, Hp, Wp):
    planes = B * C
    assert planes % _NW == 0
    ppw = planes // _NW
    W2 = 2 * Wp
    pair = 2 * W2
    CH = 48
    assert Hp % CH == 0
    cpp = Hp // CH
    gpr = Wp // _L
    in_words = CH * Wp
    out_words = CH * pair
    rows_pw = ppw * Hp
    nch = ppw * cpp
    npair = nch // 2
    assert nch % 2 == 0 and npair >= 3

    mesh = plsc.VectorSubcoreMesh(core_axis_name="c", subcore_axis_name="s")

    @functools.partial(
        pl.kernel,
        out_type=jax.ShapeDtypeStruct((planes * Hp * Wp * 4,), jnp.float32),
        mesh=mesh,
        compiler_params=pltpu.CompilerParams(needs_layout_passes=False),
        scratch_types=[
            pltpu.VMEM((in_words,), jnp.float32),
            pltpu.VMEM((in_words,), jnp.float32),
            pltpu.VMEM((in_words,), jnp.int32),
            pltpu.VMEM((in_words,), jnp.int32),
            pltpu.VMEM((out_words,), jnp.float32),
            pltpu.VMEM((out_words,), jnp.float32),
            pltpu.SemaphoreType.DMA,
            pltpu.SemaphoreType.DMA,
            pltpu.SemaphoreType.DMA,
            pltpu.SemaphoreType.DMA,
            pltpu.SemaphoreType.DMA,
            pltpu.SemaphoreType.DMA,
        ],
    )
    def unpool(f_hbm, p_hbm, out_hbm,
               fb0, fb1, pb0, pb1, ob0, ob1, sf0, sf1, sp0, sp1, so0, so1):
        wid = lax.axis_index("s") * _NUM_CORES + lax.axis_index("c")
        zero = jnp.zeros((_L,), jnp.float32)
        bufsets = ((fb0, pb0, ob0, sf0, sp0, so0),
                   (fb1, pb1, ob1, sf1, sp1, so1))

        def row0_of(ch):
            return wid * rows_pw + ch * CH

        def in_copies(ch, fb, pb, sf, sp):
            row0 = row0_of(ch)
            src = pl.ds(row0 * Wp, in_words)
            return (pltpu.make_async_copy(f_hbm.at[src], fb, sf),
                    pltpu.make_async_copy(p_hbm.at[src], pb, sp))

        def start_in(ch, fb, pb, sf, sp):
            for c in in_copies(ch, fb, pb, sf, sp):
                c.start()

        def wait_in(ch, fb, pb, sf, sp):
            for c in in_copies(ch, fb, pb, sf, sp):
                c.wait()

        def out_copy(ch, ob, so):
            row0 = row0_of(ch)
            return pltpu.make_async_copy(
                ob.at[pl.ds(0, out_words)],
                out_hbm.at[pl.ds(row0 * pair, out_words)], so)

        def compute(ch, fb, pb, ob):
            i_base = lax.rem(ch, cpp) * CH

            @plsc.parallel_loop(0, CH, unroll=1)
            def row_body(k):
                qc = pair * (i_base + k)
                rowoff = pair * k
                obp = ob.at[pl.ds(rowoff, pair)]
                for z in range(pair // _L):
                    obp[pl.ds(z * _L, _L)] = zero
                for t in range(gpr):
                    off = k * Wp + t * _L
                    pv = pb[pl.ds(off, _L)]
                    fv = fb[pl.ds(off, _L)]
                    plsc.store_scatter(obp, [pv - qc], fv)

        for b, (fb, pb, ob, sf, sp, so) in enumerate(bufsets):
            start_in(b, fb, pb, sf, sp)

        for b, (fb, pb, ob, sf, sp, so) in enumerate(bufsets):
            wait_in(b, fb, pb, sf, sp)
            compute(b, fb, pb, ob)
            out_copy(b, ob, so).start()
            start_in(b + 2, fb, pb, sf, sp)

        def mid(g, _):
            for b, (fb, pb, ob, sf, sp, so) in enumerate(bufsets):
                ch = 2 * g + b
                wait_in(ch, fb, pb, sf, sp)
                out_copy(ch - 2, ob, so).wait()
                compute(ch, fb, pb, ob)
                out_copy(ch, ob, so).start()
                start_in(ch + 2, fb, pb, sf, sp)
            return 0

        lax.fori_loop(1, npair - 1, mid, 0)

        for b, (fb, pb, ob, sf, sp, so) in enumerate(bufsets):
            ch = nch - 2 + b
            wait_in(ch, fb, pb, sf, sp)
            out_copy(ch - 2, ob, so).wait()
            compute(ch, fb, pb, ob)
            out_copy(ch, ob, so).start()

        for b, (fb, pb, ob, sf, sp, so) in enumerate(bufsets):
            out_copy(nch - 2 + b, ob, so).wait()

    return unpool


def kernel(f, provenance, size):
    B, C, Hp, Wp = f.shape
    unpool = _build_sc_unpool(B, C, Hp, Wp)
    out_flat = unpool(f.reshape(-1), provenance.reshape(-1))
    return out_flat.reshape(B, C, 2 * Hp, 2 * Wp)


# speedup vs baseline: 128.6548x; 1.0056x over previous
import functools

import jax
import jax.numpy as jnp
from jax import lax
from jax.experimental import pallas as pl
from jax.experimental.pallas import tpu as pltpu
from jax.experimental.pallas import tpu_sc as plsc

_NUM_CORES = 2
_NUM_SUBCORES = 16
_NW = _NUM_CORES * _NUM_SUBCORES
_L = 16


def _build_sc_unpool(B, C, Hp, Wp):
    planes = B * C
    assert planes % _NW == 0
    ppw = planes // _NW
    W2 = 2 * Wp
    pair = 2 * W2
    CH = 48
    assert Hp % CH == 0
    cpp = Hp // CH
    gpr = Wp // _L
    in_words = CH * Wp
    rows_pw = ppw * Hp
    R = planes * Hp
    nch = ppw * cpp
    npair = nch // 2
    assert nch % 2 == 0 and npair >= 3

    mesh = plsc.VectorSubcoreMesh(core_axis_name="c", subcore_axis_name="s")

    @functools.partial(
        pl.kernel,
        out_type=jax.ShapeDtypeStruct((R, pair), jnp.float32),
        mesh=mesh,
        compiler_params=pltpu.CompilerParams(needs_layout_passes=False),
        scratch_types=[
            pltpu.VMEM((in_words,), jnp.float32),
            pltpu.VMEM((in_words,), jnp.float32),
            pltpu.VMEM((in_words,), jnp.int32),
            pltpu.VMEM((in_words,), jnp.int32),
            pltpu.VMEM((CH, pair), jnp.float32),
            pltpu.VMEM((CH, pair), jnp.float32),
            pltpu.SemaphoreType.DMA,
            pltpu.SemaphoreType.DMA,
            pltpu.SemaphoreType.DMA,
            pltpu.SemaphoreType.DMA,
            pltpu.SemaphoreType.DMA,
            pltpu.SemaphoreType.DMA,
        ],
    )
    def unpool(f_hbm, p_hbm, out_hbm,
               fb0, fb1, pb0, pb1, ob0, ob1, sf0, sf1, sp0, sp1, so0, so1):
        wid = lax.axis_index("s") * _NUM_CORES + lax.axis_index("c")
        zero = jnp.zeros((_L,), jnp.float32)
        bufsets = ((fb0, pb0, ob0, sf0, sp0, so0),
                   (fb1, pb1, ob1, sf1, sp1, so1))

        def row0_of(ch):
            return wid * rows_pw + ch * CH

        def in_copies(ch, fb, pb, sf, sp):
            row0 = row0_of(ch)
            src = pl.ds(row0 * Wp, in_words)
            return (pltpu.make_async_copy(f_hbm.at[src], fb, sf),
                    pltpu.make_async_copy(p_hbm.at[src], pb, sp))

        def start_in(ch, fb, pb, sf, sp):
            for c in in_copies(ch, fb, pb, sf, sp):
                c.start()

        def wait_in(ch, fb, pb, sf, sp):
            for c in in_copies(ch, fb, pb, sf, sp):
                c.wait()

        def out_copy(ch, ob, so):
            row0 = row0_of(ch)
            return pltpu.make_async_copy(
                ob, out_hbm.at[pl.ds(row0, CH), :], so)

        def compute(ch, fb, pb, ob):
            i_base = lax.rem(ch, cpp) * CH

            @plsc.parallel_loop(0, CH, unroll=1)
            def row_body(k):
                qc = pair * (i_base + k)
                for z in range(pair // _L):
                    ob[k, pl.ds(z * _L, _L)] = zero
                krow = jnp.full((_L,), k, jnp.int32)
                for t in range(gpr):
                    off = k * Wp + t * _L
                    pv = pb[pl.ds(off, _L)]
                    fv = fb[pl.ds(off, _L)]
                    plsc.store_scatter(ob, [krow, pv - qc], fv)

        for b, (fb, pb, ob, sf, sp, so) in enumerate(bufsets):
            start_in(b, fb, pb, sf, sp)

        for b, (fb, pb, ob, sf, sp, so) in enumerate(bufsets):
            wait_in(b, fb, pb, sf, sp)
            compute(b, fb, pb, ob)
            out_copy(b, ob, so).start()
            start_in(b + 2, fb, pb, sf, sp)

        def mid(g, _):
            for b, (fb, pb, ob, sf, sp, so) in enumerate(bufsets):
                ch = 2 * g + b
                wait_in(ch, fb, pb, sf, sp)
                out_copy(ch - 2, ob, so).wait()
                compute(ch, fb, pb, ob)
                out_copy(ch, ob, so).start()
                start_in(ch + 2, fb, pb, sf, sp)
            return 0

        lax.fori_loop(1, npair - 1, mid, 0)

        for b, (fb, pb, ob, sf, sp, so) in enumerate(bufsets):
            ch = nch - 2 + b
            wait_in(ch, fb, pb, sf, sp)
            out_copy(ch - 2, ob, so).wait()
            compute(ch, fb, pb, ob)
            out_copy(ch, ob, so).start()

        for b, (fb, pb, ob, sf, sp, so) in enumerate(bufsets):
            out_copy(nch - 2 + b, ob, so).wait()

    return unpool


def kernel(f, provenance, size):
    B, C, Hp, Wp = f.shape
    unpool = _build_sc_unpool(B, C, Hp, Wp)
    out = unpool(f.reshape(-1), provenance.reshape(-1))
    return out.reshape(B, C, 2 * Hp, 2 * Wp)


# speedup vs baseline: 225.2733x; 1.7510x over previous
import functools

import jax
import jax.numpy as jnp
from jax import lax
from jax.experimental import pallas as pl
from jax.experimental.pallas import tpu as pltpu
from jax.experimental.pallas import tpu_sc as plsc

_NUM_CORES = 2
_NUM_SUBCORES = 16
_NW = _NUM_CORES * _NUM_SUBCORES
_L = 16


def _build_sc_unpool(B, C, Hp, Wp):
    planes = B * C
    assert planes % _NW == 0
    ppw = planes // _NW
    W2 = 2 * Wp
    pair = 2 * W2
    CH = 48
    assert Hp % CH == 0
    cpp = Hp // CH
    gpr = Wp // _L
    in_words = CH * Wp
    rows_pw = ppw * Hp
    R = planes * Hp
    nch = ppw * cpp
    npair = nch // 2
    assert nch % 2 == 0 and npair >= 3

    mesh = plsc.VectorSubcoreMesh(core_axis_name="c", subcore_axis_name="s")

    @functools.partial(
        pl.kernel,
        out_type=jax.ShapeDtypeStruct((B, C, 2 * Hp, W2), jnp.float32),
        mesh=mesh,
        compiler_params=pltpu.CompilerParams(needs_layout_passes=False),
        scratch_types=[
            pltpu.VMEM((in_words,), jnp.float32),
            pltpu.VMEM((in_words,), jnp.float32),
            pltpu.VMEM((in_words,), jnp.int32),
            pltpu.VMEM((in_words,), jnp.int32),
            pltpu.VMEM((2 * CH, W2), jnp.float32),
            pltpu.VMEM((2 * CH, W2), jnp.float32),
            pltpu.SemaphoreType.DMA,
            pltpu.SemaphoreType.DMA,
            pltpu.SemaphoreType.DMA,
            pltpu.SemaphoreType.DMA,
            pltpu.SemaphoreType.DMA,
            pltpu.SemaphoreType.DMA,
        ],
    )
    def unpool(f_hbm, p_hbm, out_hbm,
               fb0, fb1, pb0, pb1, ob0, ob1, sf0, sf1, sp0, sp1, so0, so1):
        wid = lax.axis_index("s") * _NUM_CORES + lax.axis_index("c")
        zero = jnp.zeros((_L,), jnp.float32)
        bufsets = ((fb0, pb0, ob0, sf0, sp0, so0),
                   (fb1, pb1, ob1, sf1, sp1, so1))

        def row0_of(ch):
            return wid * rows_pw + ch * CH

        def in_copies(ch, fb, pb, sf, sp):
            row0 = row0_of(ch)
            src = pl.ds(row0 * Wp, in_words)
            return (pltpu.make_async_copy(f_hbm.at[src], fb, sf),
                    pltpu.make_async_copy(p_hbm.at[src], pb, sp))

        def start_in(ch, fb, pb, sf, sp):
            for c in in_copies(ch, fb, pb, sf, sp):
                c.start()

        def wait_in(ch, fb, pb, sf, sp):
            for c in in_copies(ch, fb, pb, sf, sp):
                c.wait()

        def out_copy(ch, ob, so):
            row0 = row0_of(ch)
            p = lax.div(row0, Hp)
            i0 = lax.rem(row0, Hp)
            return pltpu.make_async_copy(
                ob,
                out_hbm.at[lax.div(p, C), lax.rem(p, C),
                           pl.ds(2 * i0, 2 * CH), :],
                so)

        def compute(ch, fb, pb, ob):
            i_base = lax.rem(ch, cpp) * CH

            @plsc.parallel_loop(0, CH, unroll=1)
            def row_body(k):
                qc = pair * (i_base + k)
                for r in range(2):
                    for z in range(W2 // _L):
                        ob[2 * k + r, pl.ds(z * _L, _L)] = zero
                for t in range(gpr):
                    off = k * Wp + t * _L
                    pv = pb[pl.ds(off, _L)]
                    fv = fb[pl.ds(off, _L)]
                    q = pv - qc
                    di = (q >= W2).astype(jnp.int32)
                    plsc.store_scatter(ob, [2 * k + di, q - W2 * di], fv)

        for b, (fb, pb, ob, sf, sp, so) in enumerate(bufsets):
            start_in(b, fb, pb, sf, sp)

        for b, (fb, pb, ob, sf, sp, so) in enumerate(bufsets):
            wait_in(b, fb, pb, sf, sp)
            compute(b, fb, pb, ob)
            out_copy(b, ob, so).start()
            start_in(b + 2, fb, pb, sf, sp)

        def mid(g, _):
            for b, (fb, pb, ob, sf, sp, so) in enumerate(bufsets):
                ch = 2 * g + b
                wait_in(ch, fb, pb, sf, sp)
                out_copy(ch - 2, ob, so).wait()
                compute(ch, fb, pb, ob)
                out_copy(ch, ob, so).start()
                start_in(ch + 2, fb, pb, sf, sp)
            return 0

        lax.fori_loop(1, npair - 1, mid, 0)

        for b, (fb, pb, ob, sf, sp, so) in enumerate(bufsets):
            ch = nch - 2 + b
            wait_in(ch, fb, pb, sf, sp)
            out_copy(ch - 2, ob, so).wait()
            compute(ch, fb, pb, ob)
            out_copy(ch, ob, so).start()

        for b, (fb, pb, ob, sf, sp, so) in enumerate(bufsets):
            out_copy(nch - 2 + b, ob, so).wait()

    return unpool


def kernel(f, provenance, size):
    B, C, Hp, Wp = f.shape
    unpool = _build_sc_unpool(B, C, Hp, Wp)
    return unpool(f.reshape(-1), provenance.reshape(-1))
